# baseline (speedup 1.0000x reference)
import functools

import jax
import jax.numpy as jnp
from jax import lax
from jax.experimental import pallas as pl
from jax.experimental.pallas import tpu as pltpu
from jax.experimental.pallas import tpu_sc as plsc

_B = 1024
_NU = 128
_NP = 512
_TEMP = 0.3
_LOW = 0.4
_HIGH = 0.7
_MARGIN = 0.1
_LAMBDA_ORTHO = 0.1
_CHUNK = 128
_NCHUNK = _B // _CHUNK
_L = 16
_NVEC = _B // _L


def _tc1_body(u_ref, p_ref, ut_ref, dist_ref, ortho_ref):
    diff = u_ref[...] - p_ref[...] + 1e-6
    dist_ref[...] = jnp.sqrt(jnp.sum(diff * diff, axis=1, keepdims=True))
    gram = jnp.dot(ut_ref[...], p_ref[...], preferred_element_type=jnp.float32)
    ortho_ref[...] = jnp.reshape(jnp.mean(jnp.abs(gram)), (1, 1))


_sc_mesh = plsc.VectorSubcoreMesh(core_axis_name="c", subcore_axis_name="s")


@functools.partial(
    pl.kernel,
    out_type=[
        jax.ShapeDtypeStruct((_B,), jnp.float32),
        jax.ShapeDtypeStruct((4 * _NU,), jnp.float32),
        jax.ShapeDtypeStruct((_L,), jnp.float32),
        jax.ShapeDtypeStruct((_L,), jnp.float32),
    ],
    mesh=_sc_mesh,
    compiler_params=pltpu.CompilerParams(needs_layout_passes=False),
    scratch_types=[
        pltpu.VMEM((_B,), jnp.int32),
        pltpu.VMEM((_B,), jnp.int32),
        pltpu.VMEM((_B,), jnp.float32),
        pltpu.VMEM((_B,), jnp.float32),
        pltpu.VMEM((_NU * _NP,), jnp.int32),
        pltpu.VMEM((_NP,), jnp.int32),
        pltpu.VMEM((_NU,), jnp.int32),
        pltpu.VMEM((_B,), jnp.float32),
        pltpu.VMEM((4 * _NU,), jnp.float32),
        pltpu.VMEM((_L,), jnp.float32),
        pltpu.VMEM((_L,), jnp.float32),
    ],
)
def _sc_sparse(uid_hbm, pid_hbm, t_hbm, d_hbm,
               win_out, acc_out, np_out, u_out,
               uid_v, pid_v, t_v, d_v, ws, pws, uws, win_v, acc_v, np_v, u_v):
    is_leader = (lax.axis_index("c") == 0) & (lax.axis_index("s") == 0)

    @pl.when(is_leader)
    def _():
        pltpu.sync_copy(uid_hbm, uid_v)
        pltpu.sync_copy(pid_hbm, pid_v)
        pltpu.sync_copy(t_hbm, t_v)
        pltpu.sync_copy(d_hbm, d_v)

        lanes = lax.iota(jnp.int32, _L)

        def scatter_pass(i, c):
            base = i * _L
            ivec = lanes + base
            u16 = uid_v[pl.ds(base, _L)]
            p16 = pid_v[pl.ds(base, _L)]
            plsc.store_scatter(ws, [u16 * _NP + p16], ivec)
            plsc.store_scatter(pws, [p16], ivec)
            plsc.store_scatter(uws, [u16], ivec)
            return c

        lax.fori_loop(0, _NVEC, scatter_pass, 0)

        def zero_acc(j, c):
            acc_v[pl.ds(j * _L, _L)] = jnp.zeros((_L,), jnp.float32)
            return c

        lax.fori_loop(0, (4 * _NU) // _L, zero_acc, 0)

        def gather_pass(i, carry):
            npacc, uacc = carry
            base = i * _L
            ivec = lanes + base
            u16 = uid_v[pl.ds(base, _L)]
            p16 = pid_v[pl.ds(base, _L)]
            t16 = t_v[pl.ds(base, _L)]
            d16 = d_v[pl.ds(base, _L)]
            wkey = plsc.load_gather(ws, [u16 * _NP + p16])
            wf = jnp.where(wkey == ivec, 1.0, 0.0)
            win_v[pl.ds(base, _L)] = wf
            e = jnp.exp(-d16 / _TEMP)
            pos = jnp.where(t16 > _HIGH, 1.0, 0.0)
            wgt = jnp.where(t16 > _HIGH, 1.0,
                            jnp.where(t16 < _LOW, 1.5, 0.3))
            plsc.addupdate_scatter(acc_v, [u16], wf)
            plsc.addupdate_scatter(acc_v, [u16 + _NU], wf * wgt * e)
            plsc.addupdate_scatter(acc_v, [u16 + 2 * _NU], wf * pos * e)
            plsc.addupdate_scatter(acc_v, [u16 + 3 * _NU], wf * pos)
            pw = plsc.load_gather(pws, [p16])
            npacc = npacc + jnp.where(pw == ivec, 1.0, 0.0)
            uw = plsc.load_gather(uws, [u16])
            uacc = uacc + jnp.where(uw == ivec, 1.0, 0.0)
            return npacc, uacc

        z16 = jnp.zeros((_L,), jnp.float32)
        npacc, uacc = lax.fori_loop(0, _NVEC, gather_pass, (z16, z16))
        np_v[...] = npacc
        u_v[...] = uacc
        pltpu.sync_copy(win_v, win_out)
        pltpu.sync_copy(acc_v, acc_out)
        pltpu.sync_copy(np_v, np_out)
        pltpu.sync_copy(u_v, u_out)


def _tc2_body(t_row_ref, t_col_ref, uid_row_ref, uid_col_ref,
              drow_ref, dcol_ref, wrow_ref, wcol_ref,
              acc_ref, np_ref, u_ref, ortho_ref, out_ref):
    f32 = jnp.float32
    t_row = t_row_ref[...]
    uid_row = uid_row_ref[...]
    dist_row = drow_ref[...]
    winner_row = wrow_ref[...]
    nP = jnp.sum(np_ref[...])
    Ucnt = jnp.sum(u_ref[...])

    def pass_b(ib, hacc):
        base = ib * _CHUNK
        uc = uid_col_ref[pl.ds(base, _CHUNK), :]
        tc = t_col_ref[pl.ds(base, _CHUNK), :]
        dc = dcol_ref[pl.ds(base, _CHUNK), :]
        wc = wcol_ref[pl.ds(base, _CHUNK), :]
        i_g = lax.broadcasted_iota(jnp.int32, (_CHUNK, _B), 0) + base
        j_g = lax.broadcasted_iota(jnp.int32, (_CHUNK, _B), 1)
        term = jax.nn.relu(jnp.sign(t_row - tc) * (dc - dist_row) + _MARGIN)
        mask = ((uc == uid_row) & (j_g > i_g)
                & (tc != t_row) & (tc > 0.0) & (t_row > 0.0))
        contrib = term * mask.astype(f32) * wc * winner_row
        return hacc + jnp.sum(contrib, axis=0, keepdims=True)

    hinge_vec = lax.fori_loop(0, _NCHUNK, pass_b, jnp.zeros((1, _B), f32))
    hinge = jnp.sum(hinge_vec) / Ucnt

    k_u = acc_ref[0:1, :]
    sum_we = acc_ref[1:2, :]
    num = acc_ref[2:3, :]
    npos = acc_ref[3:4, :]
    denom = 0.3 * (nP - k_u) + sum_we + 1e-8
    valid = (npos > 0.0).astype(f32)
    num_safe = jnp.where(npos > 0.0, num, denom)
    lpu = -jnp.log(num_safe / denom)
    n_valid = jnp.sum(valid)
    nce = jnp.where(n_valid > 0.0,
                    jnp.sum(lpu * valid) / jnp.maximum(n_valid, 1.0), 0.0)

    total = nce + hinge + ortho_ref[0, 0] * _LAMBDA_ORTHO
    out_ref[...] = jnp.reshape(total, (1, 1))


@jax.jit
def _run(u_emb, p_emb, t, uid, pid):
    dist, ortho = pl.pallas_call(
        _tc1_body,
        out_shape=[jax.ShapeDtypeStruct((_B, 1), jnp.float32),
                   jax.ShapeDtypeStruct((1, 1), jnp.float32)],
    )(u_emb, p_emb, u_emb.T)

    dist_flat = dist.reshape(_B)
    win, acc, np_p, u_p = _sc_sparse(uid, pid, t, dist_flat)

    out = pl.pallas_call(
        _tc2_body,
        out_shape=jax.ShapeDtypeStruct((1, 1), jnp.float32),
    )(t.reshape(1, _B), t.reshape(_B, 1),
      uid.reshape(1, _B), uid.reshape(_B, 1),
      dist.reshape(1, _B), dist,
      win.reshape(1, _B), win.reshape(_B, 1),
      acc.reshape(4, _NU), np_p.reshape(1, _L), u_p.reshape(1, _L),
      ortho)
    return out[0, 0]


def kernel(u_emb, p_emb, p_views, t, user_ids, prop_ids):
    del p_views
    return _run(u_emb, p_emb, t.astype(jnp.float32),
                user_ids.astype(jnp.int32), prop_ids.astype(jnp.int32))


# speedup vs baseline: 1.1644x; 1.1644x over previous
import functools

import jax
import jax.numpy as jnp
from jax import lax
from jax.experimental import pallas as pl
from jax.experimental.pallas import tpu as pltpu
from jax.experimental.pallas import tpu_sc as plsc

_B = 1024
_NU = 128
_NP = 512
_TEMP = 0.3
_LOW = 0.4
_HIGH = 0.7
_MARGIN = 0.1
_LAMBDA_ORTHO = 0.1
_CHUNK = 128
_NCHUNK = _B // _CHUNK
_L = 16
_NVEC = _B // _L


_sc_mesh = plsc.VectorSubcoreMesh(core_axis_name="c", subcore_axis_name="s")


@functools.partial(
    pl.kernel,
    out_type=[
        jax.ShapeDtypeStruct((_B,), jnp.float32),
        jax.ShapeDtypeStruct((_L,), jnp.float32),
        jax.ShapeDtypeStruct((_L,), jnp.float32),
    ],
    mesh=_sc_mesh,
    compiler_params=pltpu.CompilerParams(needs_layout_passes=False),
    scratch_types=[
        pltpu.VMEM((_B,), jnp.int32),
        pltpu.VMEM((_B,), jnp.int32),
        pltpu.VMEM((_NU * _NP,), jnp.int32),
        pltpu.VMEM((_NP,), jnp.int32),
        pltpu.VMEM((_NU,), jnp.int32),
        pltpu.VMEM((_B,), jnp.float32),
        pltpu.VMEM((_L,), jnp.float32),
        pltpu.VMEM((_L,), jnp.float32),
    ],
)
def _sc_sparse(uid_hbm, pid_hbm,
               win_out, np_out, u_out,
               uid_v, pid_v, ws, pws, uws, win_v, np_v, u_v):
    is_leader = (lax.axis_index("c") == 0) & (lax.axis_index("s") == 0)

    @pl.when(is_leader)
    def _():
        pltpu.sync_copy(uid_hbm, uid_v)
        pltpu.sync_copy(pid_hbm, pid_v)

        lanes = lax.iota(jnp.int32, _L)

        def scatter_pass(i, c):
            base = i * _L
            ivec = lanes + base
            u16 = uid_v[pl.ds(base, _L)]
            p16 = pid_v[pl.ds(base, _L)]
            plsc.store_scatter(ws, [u16 * _NP + p16], ivec)
            plsc.store_scatter(pws, [p16], ivec)
            plsc.store_scatter(uws, [u16], ivec)
            return c

        lax.fori_loop(0, _NVEC, scatter_pass, 0)

        def gather_pass(i, carry):
            npacc, uacc = carry
            base = i * _L
            ivec = lanes + base
            u16 = uid_v[pl.ds(base, _L)]
            p16 = pid_v[pl.ds(base, _L)]
            wkey = plsc.load_gather(ws, [u16 * _NP + p16])
            win_v[pl.ds(base, _L)] = jnp.where(wkey == ivec, 1.0, 0.0)
            pw = plsc.load_gather(pws, [p16])
            npacc = npacc + jnp.where(pw == ivec, 1.0, 0.0)
            uw = plsc.load_gather(uws, [u16])
            uacc = uacc + jnp.where(uw == ivec, 1.0, 0.0)
            return npacc, uacc

        z16 = jnp.zeros((_L,), jnp.float32)
        npacc, uacc = lax.fori_loop(0, _NVEC, gather_pass, (z16, z16))
        np_v[...] = npacc
        u_v[...] = uacc
        pltpu.sync_copy(win_v, win_out)
        pltpu.sync_copy(np_v, np_out)
        pltpu.sync_copy(u_v, u_out)


def _tc_body(u_ref, p_ref, ut_ref, pt_ref, t_row_ref, t_col_ref,
             uid_row_ref, uid_col_ref, wrow_ref, wcol_ref,
             np_ref, ucnt_ref, out_ref, dcol_ref):
    f32 = jnp.float32
    t_row = t_row_ref[...]
    uid_row = uid_row_ref[...]
    winner_row = wrow_ref[...]
    nP = jnp.sum(np_ref[...])
    Ucnt = jnp.sum(ucnt_ref[...])

    diff = u_ref[...] - p_ref[...] + 1e-6
    dcol_ref[...] = jnp.sqrt(jnp.sum(diff * diff, axis=1, keepdims=True))
    difft = ut_ref[...] - pt_ref[...] + 1e-6
    dist_row = jnp.sqrt(jnp.sum(difft * difft, axis=0, keepdims=True))

    def pass_b(ib, hacc):
        base = ib * _CHUNK
        uc = uid_col_ref[pl.ds(base, _CHUNK), :]
        tc = t_col_ref[pl.ds(base, _CHUNK), :]
        dc = dcol_ref[pl.ds(base, _CHUNK), :]
        wc = wcol_ref[pl.ds(base, _CHUNK), :]
        i_g = lax.broadcasted_iota(jnp.int32, (_CHUNK, _B), 0) + base
        j_g = lax.broadcasted_iota(jnp.int32, (_CHUNK, _B), 1)
        term = jax.nn.relu(jnp.sign(t_row - tc) * (dc - dist_row) + _MARGIN)
        mask = ((uc == uid_row) & (j_g > i_g)
                & (tc != t_row) & (tc > 0.0) & (t_row > 0.0))
        contrib = term * mask.astype(f32) * wc * winner_row
        return hacc + jnp.sum(contrib, axis=0, keepdims=True)

    hinge_vec = lax.fori_loop(0, _NCHUNK, pass_b, jnp.zeros((1, _B), f32))
    hinge = jnp.sum(hinge_vec) / Ucnt

    e_row = jnp.exp(-dist_row / _TEMP)
    pos_row = (t_row > _HIGH).astype(f32)
    w_row = jnp.where(t_row > _HIGH, 1.0,
                      jnp.where(t_row < _LOW, 1.5, 0.3))
    onehot = (lax.broadcasted_iota(jnp.int32, (_NU, _B), 0)
              == uid_row).astype(f32)
    k_u = jnp.sum(onehot * winner_row, axis=1, keepdims=True)
    sum_we = jnp.sum(onehot * (winner_row * w_row * e_row), axis=1, keepdims=True)
    num = jnp.sum(onehot * (winner_row * pos_row * e_row), axis=1, keepdims=True)
    npos = jnp.sum(onehot * (winner_row * pos_row), axis=1, keepdims=True)
    denom = 0.3 * (nP - k_u) + sum_we + 1e-8
    valid = (npos > 0.0).astype(f32)
    num_safe = jnp.where(npos > 0.0, num, denom)
    lpu = -jnp.log(num_safe / denom)
    n_valid = jnp.sum(valid)
    nce = jnp.where(n_valid > 0.0,
                    jnp.sum(lpu * valid) / jnp.maximum(n_valid, 1.0), 0.0)

    gram = jnp.dot(ut_ref[...], p_ref[...], preferred_element_type=f32)
    ortho = jnp.mean(jnp.abs(gram))

    total = nce + hinge + ortho * _LAMBDA_ORTHO
    out_ref[...] = jnp.reshape(total, (1, 1))


@jax.jit
def _run(u_emb, p_emb, t, uid, pid):
    win, np_p, u_p = _sc_sparse(uid, pid)

    out = pl.pallas_call(
        _tc_body,
        out_shape=jax.ShapeDtypeStruct((1, 1), jnp.float32),
        scratch_shapes=[pltpu.VMEM((_B, 1), jnp.float32)],
    )(u_emb, p_emb, u_emb.T, p_emb.T,
      t.reshape(1, _B), t.reshape(_B, 1),
      uid.reshape(1, _B), uid.reshape(_B, 1),
      win.reshape(1, _B), win.reshape(_B, 1),
      np_p.reshape(1, _L), u_p.reshape(1, _L))
    return out[0, 0]


def kernel(u_emb, p_emb, p_views, t, user_ids, prop_ids):
    del p_views
    return _run(u_emb, p_emb, t.astype(jnp.float32),
                user_ids.astype(jnp.int32), prop_ids.astype(jnp.int32))


# speedup vs baseline: 1.2627x; 1.0845x over previous
import functools

import jax
import jax.numpy as jnp
from jax import lax
from jax.experimental import pallas as pl
from jax.experimental.pallas import tpu as pltpu
from jax.experimental.pallas import tpu_sc as plsc

_B = 1024
_NU = 128
_NP = 512
_TEMP = 0.3
_LOW = 0.4
_HIGH = 0.7
_MARGIN = 0.1
_LAMBDA_ORTHO = 0.1
_CHUNK = 128
_NCHUNK = _B // _CHUNK
_L = 16
_NVEC = _B // _L


_sc_mesh = plsc.VectorSubcoreMesh(core_axis_name="c", subcore_axis_name="s")


@functools.partial(
    pl.kernel,
    out_type=[
        jax.ShapeDtypeStruct((_B,), jnp.float32),
        jax.ShapeDtypeStruct((_L,), jnp.float32),
        jax.ShapeDtypeStruct((_L,), jnp.float32),
    ],
    mesh=_sc_mesh,
    compiler_params=pltpu.CompilerParams(needs_layout_passes=False),
    scratch_types=[
        pltpu.VMEM((_B,), jnp.int32),
        pltpu.VMEM((_B,), jnp.int32),
        pltpu.VMEM((_NU * _NP,), jnp.int32),
        pltpu.VMEM((_NP,), jnp.int32),
        pltpu.VMEM((_NU,), jnp.int32),
        pltpu.VMEM((_B,), jnp.float32),
        pltpu.VMEM((_L,), jnp.float32),
        pltpu.VMEM((_L,), jnp.float32),
    ],
)
def _sc_sparse(uid_hbm, pid_hbm,
               win_out, np_out, u_out,
               uid_v, pid_v, ws, pws, uws, win_v, np_v, u_v):
    is_leader = (lax.axis_index("c") == 0) & (lax.axis_index("s") == 0)

    @pl.when(is_leader)
    def _():
        pltpu.sync_copy(uid_hbm, uid_v)
        pltpu.sync_copy(pid_hbm, pid_v)

        lanes = lax.iota(jnp.int32, _L)

        def scatter_pass(i, c):
            base = i * _L
            ivec = lanes + base
            u16 = uid_v[pl.ds(base, _L)]
            p16 = pid_v[pl.ds(base, _L)]
            plsc.store_scatter(ws, [u16 * _NP + p16], ivec)
            plsc.store_scatter(pws, [p16], ivec)
            plsc.store_scatter(uws, [u16], ivec)
            return c

        lax.fori_loop(0, _NVEC, scatter_pass, 0)

        def gather_pass(i, carry):
            npacc, uacc = carry
            base = i * _L
            ivec = lanes + base
            u16 = uid_v[pl.ds(base, _L)]
            p16 = pid_v[pl.ds(base, _L)]
            wkey = plsc.load_gather(ws, [u16 * _NP + p16])
            win_v[pl.ds(base, _L)] = jnp.where(wkey == ivec, 1.0, 0.0)
            pw = plsc.load_gather(pws, [p16])
            npacc = npacc + jnp.where(pw == ivec, 1.0, 0.0)
            uw = plsc.load_gather(uws, [u16])
            uacc = uacc + jnp.where(uw == ivec, 1.0, 0.0)
            return npacc, uacc

        z16 = jnp.zeros((_L,), jnp.float32)
        npacc, uacc = lax.fori_loop(0, _NVEC, gather_pass, (z16, z16))
        np_v[...] = npacc
        u_v[...] = uacc
        pltpu.sync_copy(win_v, win_out)
        pltpu.sync_copy(np_v, np_out)
        pltpu.sync_copy(u_v, u_out)


def _tc_body(u_ref, p_ref, t_row_ref, t_col_ref,
             uid_row_ref, uid_col_ref, wrow_ref, wcol_ref,
             np_ref, ucnt_ref, out_ref, dcol_ref):
    f32 = jnp.float32
    t_row = t_row_ref[...]
    uid_row = uid_row_ref[...]
    winner_row = wrow_ref[...]
    nP = jnp.sum(np_ref[...])
    Ucnt = jnp.sum(ucnt_ref[...])

    diff = u_ref[...] - p_ref[...] + 1e-6
    dist_col = jnp.sqrt(jnp.sum(diff * diff, axis=1, keepdims=True))
    dcol_ref[...] = dist_col
    dist_row = jnp.transpose(dist_col)

    row_ok = winner_row * (t_row > 0.0).astype(f32)

    hinge_sum = jnp.zeros((), f32)
    for ib in range(_NCHUNK):
        base = ib * _CHUNK
        width = _B - base
        uc = uid_col_ref[pl.ds(base, _CHUNK), :]
        tc = t_col_ref[pl.ds(base, _CHUNK), :]
        dc = dcol_ref[pl.ds(base, _CHUNK), :]
        wc = wcol_ref[pl.ds(base, _CHUNK), :]
        tr = t_row[:, base:]
        ur = uid_row[:, base:]
        dr = dist_row[:, base:]
        rk = row_ok[:, base:]
        i_l = lax.broadcasted_iota(jnp.int32, (_CHUNK, width), 0)
        j_l = lax.broadcasted_iota(jnp.int32, (_CHUNK, width), 1)
        term = jax.nn.relu(jnp.sign(tr - tc) * (dc - dr) + _MARGIN)
        mask = (uc == ur) & (j_l > i_l) & (tc != tr)
        contrib = term * mask.astype(f32) * (wc * (tc > 0.0).astype(f32)) * rk
        hinge_sum = hinge_sum + jnp.sum(contrib)
    hinge = hinge_sum / Ucnt

    e_row = jnp.exp(-dist_row / _TEMP)
    pos_row = (t_row > _HIGH).astype(f32)
    w_row = jnp.where(t_row > _HIGH, 1.0,
                      jnp.where(t_row < _LOW, 1.5, 0.3))
    onehot = (lax.broadcasted_iota(jnp.int32, (_NU, _B), 0)
              == uid_row).astype(f32)
    k_u = jnp.sum(onehot * winner_row, axis=1, keepdims=True)
    sum_we = jnp.sum(onehot * (winner_row * w_row * e_row), axis=1, keepdims=True)
    num = jnp.sum(onehot * (winner_row * pos_row * e_row), axis=1, keepdims=True)
    npos = jnp.sum(onehot * (winner_row * pos_row), axis=1, keepdims=True)
    denom = 0.3 * (nP - k_u) + sum_we + 1e-8
    valid = (npos > 0.0).astype(f32)
    num_safe = jnp.where(npos > 0.0, num, denom)
    lpu = -jnp.log(num_safe / denom)
    n_valid = jnp.sum(valid)
    nce = jnp.where(n_valid > 0.0,
                    jnp.sum(lpu * valid) / jnp.maximum(n_valid, 1.0), 0.0)

    gram = lax.dot_general(u_ref[...], p_ref[...],
                           dimension_numbers=(((0,), (0,)), ((), ())),
                           preferred_element_type=f32)
    ortho = jnp.mean(jnp.abs(gram))

    total = nce + hinge + ortho * _LAMBDA_ORTHO
    out_ref[...] = jnp.reshape(total, (1, 1))


@jax.jit
def _run(u_emb, p_emb, t, uid, pid):
    win, np_p, u_p = _sc_sparse(uid, pid)

    out = pl.pallas_call(
        _tc_body,
        out_shape=jax.ShapeDtypeStruct((1, 1), jnp.float32),
        scratch_shapes=[pltpu.VMEM((_B, 1), jnp.float32)],
    )(u_emb, p_emb,
      t.reshape(1, _B), t.reshape(_B, 1),
      uid.reshape(1, _B), uid.reshape(_B, 1),
      win.reshape(1, _B), win.reshape(_B, 1),
      np_p.reshape(1, _L), u_p.reshape(1, _L))
    return out[0, 0]


def kernel(u_emb, p_emb, p_views, t, user_ids, prop_ids):
    del p_views
    return _run(u_emb, p_emb, t.astype(jnp.float32),
                user_ids.astype(jnp.int32), prop_ids.astype(jnp.int32))


# speedup vs baseline: 1.3603x; 1.0772x over previous
import functools

import jax
import jax.numpy as jnp
from jax import lax
from jax.experimental import pallas as pl
from jax.experimental.pallas import tpu as pltpu
from jax.experimental.pallas import tpu_sc as plsc

_B = 1024
_NU = 128
_NP = 512
_TEMP = 0.3
_LOW = 0.4
_HIGH = 0.7
_MARGIN = 0.1
_LAMBDA_ORTHO = 0.1
_CHUNK = 128
_NCHUNK = _B // _CHUNK
_L = 16
_NVEC = _B // _L


_sc_mesh = plsc.VectorSubcoreMesh(core_axis_name="c", subcore_axis_name="s")


@functools.partial(
    pl.kernel,
    out_type=jax.ShapeDtypeStruct((_B,), jnp.float32),
    mesh=_sc_mesh,
    compiler_params=pltpu.CompilerParams(needs_layout_passes=False),
    scratch_types=[
        pltpu.VMEM((_B,), jnp.int32),
        pltpu.VMEM((_B,), jnp.int32),
        pltpu.VMEM((_NU * _NP,), jnp.int32),
        pltpu.VMEM((_B,), jnp.float32),
    ],
)
def _sc_dedup(uid_hbm, pid_hbm, win_out, uid_v, pid_v, ws, win_v):
    is_leader = (lax.axis_index("c") == 0) & (lax.axis_index("s") == 0)

    @pl.when(is_leader)
    def _():
        pltpu.sync_copy(uid_hbm, uid_v)
        pltpu.sync_copy(pid_hbm, pid_v)

        lanes = lax.iota(jnp.int32, _L)

        def scatter_pass(i, c):
            base = i * _L
            u16 = uid_v[pl.ds(base, _L)]
            p16 = pid_v[pl.ds(base, _L)]
            plsc.store_scatter(ws, [u16 * _NP + p16], lanes + base)
            return c

        lax.fori_loop(0, _NVEC, scatter_pass, 0)

        def gather_pass(i, c):
            base = i * _L
            u16 = uid_v[pl.ds(base, _L)]
            p16 = pid_v[pl.ds(base, _L)]
            wkey = plsc.load_gather(ws, [u16 * _NP + p16])
            win_v[pl.ds(base, _L)] = jnp.where(wkey == lanes + base, 1.0, 0.0)
            return c

        lax.fori_loop(0, _NVEC, gather_pass, 0)
        pltpu.sync_copy(win_v, win_out)


def _tca_body(u_ref, p_ref, dcol_ref, drow_ref, ortho_ref):
    f32 = jnp.float32
    diff = u_ref[...] - p_ref[...] + 1e-6
    dist_col = jnp.sqrt(jnp.sum(diff * diff, axis=1, keepdims=True))
    dcol_ref[...] = dist_col
    drow_ref[...] = jnp.transpose(dist_col)
    gram = lax.dot_general(u_ref[...], p_ref[...],
                           dimension_numbers=(((0,), (0,)), ((), ())),
                           preferred_element_type=f32)
    ortho_ref[...] = jnp.reshape(jnp.mean(jnp.abs(gram)), (1, 1))


def _tcb_body(t_row_ref, uid_row_ref, pid_row_ref, wrow_ref, dcol_ref,
              drow_ref, ortho_ref, out_ref, tcol_ref, ucol_ref, wcol_ref):
    f32 = jnp.float32
    t_row = t_row_ref[...]
    uid_row = uid_row_ref[...]
    pid_row = pid_row_ref[...]
    winner_row = wrow_ref[...]
    dist_row = drow_ref[...]

    tcol_ref[...] = jnp.transpose(t_row)
    ucol_ref[...] = jnp.transpose(uid_row)
    wcol_ref[...] = jnp.transpose(winner_row)

    onehot_u = (lax.broadcasted_iota(jnp.int32, (_NU, _B), 0)
                == uid_row).astype(f32)
    present_u = jnp.max(onehot_u, axis=1, keepdims=True)
    Ucnt = jnp.sum(present_u)
    onehot_p = (lax.broadcasted_iota(jnp.int32, (_NP, _B), 0)
                == pid_row).astype(f32)
    nP = jnp.sum(jnp.max(onehot_p, axis=1))

    row_ok = winner_row * (t_row > 0.0).astype(f32)

    hinge_sum = jnp.zeros((), f32)
    for ib in range(_NCHUNK):
        base = ib * _CHUNK
        width = _B - base
        uc = ucol_ref[pl.ds(base, _CHUNK), :]
        tc = tcol_ref[pl.ds(base, _CHUNK), :]
        dc = dcol_ref[pl.ds(base, _CHUNK), :]
        wc = wcol_ref[pl.ds(base, _CHUNK), :]
        tr = t_row[:, base:]
        ur = uid_row[:, base:]
        dr = dist_row[:, base:]
        rk = row_ok[:, base:]
        i_l = lax.broadcasted_iota(jnp.int32, (_CHUNK, width), 0)
        j_l = lax.broadcasted_iota(jnp.int32, (_CHUNK, width), 1)
        term = jax.nn.relu(jnp.sign(tr - tc) * (dc - dr) + _MARGIN)
        mask = (uc == ur) & (j_l > i_l) & (tc != tr)
        contrib = term * mask.astype(f32) * (wc * (tc > 0.0).astype(f32)) * rk
        hinge_sum = hinge_sum + jnp.sum(contrib)
    hinge = hinge_sum / Ucnt

    e_row = jnp.exp(-dist_row / _TEMP)
    pos_row = (t_row > _HIGH).astype(f32)
    w_row = jnp.where(t_row > _HIGH, 1.0,
                      jnp.where(t_row < _LOW, 1.5, 0.3))
    k_u = jnp.sum(onehot_u * winner_row, axis=1, keepdims=True)
    sum_we = jnp.sum(onehot_u * (winner_row * w_row * e_row), axis=1,
                     keepdims=True)
    num = jnp.sum(onehot_u * (winner_row * pos_row * e_row), axis=1,
                  keepdims=True)
    npos = jnp.sum(onehot_u * (winner_row * pos_row), axis=1, keepdims=True)
    denom = 0.3 * (nP - k_u) + sum_we + 1e-8
    valid = (npos > 0.0).astype(f32)
    num_safe = jnp.where(npos > 0.0, num, denom)
    lpu = -jnp.log(num_safe / denom)
    n_valid = jnp.sum(valid)
    nce = jnp.where(n_valid > 0.0,
                    jnp.sum(lpu * valid) / jnp.maximum(n_valid, 1.0), 0.0)

    total = nce + hinge + ortho_ref[0, 0] * _LAMBDA_ORTHO
    out_ref[...] = jnp.reshape(total, (1, 1))


@jax.jit
def _run(u_emb, p_emb, t, uid, pid):
    win = _sc_dedup(uid, pid)

    dcol, drow, ortho = pl.pallas_call(
        _tca_body,
        out_shape=[jax.ShapeDtypeStruct((_B, 1), jnp.float32),
                   jax.ShapeDtypeStruct((1, _B), jnp.float32),
                   jax.ShapeDtypeStruct((1, 1), jnp.float32)],
    )(u_emb, p_emb)

    out = pl.pallas_call(
        _tcb_body,
        out_shape=jax.ShapeDtypeStruct((1, 1), jnp.float32),
        scratch_shapes=[pltpu.VMEM((_B, 1), jnp.float32),
                        pltpu.VMEM((_B, 1), jnp.int32),
                        pltpu.VMEM((_B, 1), jnp.float32)],
    )(t.reshape(1, _B), uid.reshape(1, _B), pid.reshape(1, _B),
      win.reshape(1, _B), dcol, drow, ortho)
    return out[0, 0]


def kernel(u_emb, p_emb, p_views, t, user_ids, prop_ids):
    del p_views
    return _run(u_emb, p_emb, t.astype(jnp.float32),
                user_ids.astype(jnp.int32), prop_ids.astype(jnp.int32))
